# baseline (speedup 1.0000x reference)
import jax
import jax.numpy as jnp
from jax import lax
from jax.experimental import pallas as pl
from jax.experimental.pallas import tpu as pltpu
from jax.experimental.pallas import tpu_sc as plsc

_NUM_SC_CORES = 2
_NUM_SC_SUBCORES = 16


def _tc_body(y_hbm, x_hbm, lt_ref, gt_ref, w_ref, b_ref, vtw_hbm, vtb_ref,
             val_ref, comb_ref, ab_ref, ew_ref, bf_ref, vtwt_ref,
             y_s, x_s, vtw_s, sem):
    b = pl.program_id(0)
    g = pl.program_id(1)
    L, D = comb_ref.shape
    DY = y_s.shape[1]
    DX, K = w_ref.shape
    F = DX * K
    G = val_ref.shape[1] // L

    @pl.when((b == 0) & (g == 0))
    def _prep_weights():
        pltpu.async_copy(vtw_hbm, vtw_s, sem).wait()
        fidx = lax.broadcasted_iota(jnp.int32, (DX, F), 1)
        didx = lax.broadcasted_iota(jnp.int32, (DX, F), 0)
        e = (fidx // K == didx).astype(jnp.float32)
        wt = jnp.concatenate([w_ref[...]] * DX, axis=1)
        bt = jnp.concatenate([b_ref[...]] * DX, axis=1)
        ew_ref[...] = e * wt
        bf_ref[...] = jnp.sum(e * bt, axis=0, keepdims=True)
        eye = (lax.broadcasted_iota(jnp.int32, (D, D), 0)
               == lax.broadcasted_iota(jnp.int32, (D, D), 1)
               ).astype(jnp.float32)
        vtwt_ref[0:F + 1, :] = lax.dot_general(
            vtw_s[...], eye, (((0,), (0,)), ((), ())),
            preferred_element_type=jnp.float32)

    @pl.when(g == 0)
    def _compute_combined():
        cp_y = pltpu.async_copy(y_hbm.at[b], y_s, sem)
        cp_x = pltpu.async_copy(x_hbm.at[b], x_s, sem)
        cp_x.wait()
        xb = jnp.where(jnp.isnan(x_s[...]), 0.0, x_s[...])
        xa = jnp.dot(xb, ew_ref[...],
                     preferred_element_type=jnp.float32) + bf_ref[...]
        ksel = (lax.broadcasted_iota(jnp.int32, (1, F), 1) % K) > 0
        feats = jnp.where(ksel, jnp.sin(xa), xa)
        tp = jnp.dot(feats, vtwt_ref[0:F, :],
                     preferred_element_type=jnp.float32)
        comb_ref[...] = (tp + lt_ref[...] + vtb_ref[...] + gt_ref[1:2, :])
        cp_y.wait()
        y2d = y_s[...]
        nan2d = jnp.isnan(y2d)
        ab_ref[:, :DY] = jnp.where(nan2d, 0.0, y2d)
        ab_ref[:, DY:] = nan2d.astype(jnp.float32)

    wy = vtwt_ref[F:F + 1, :]
    delta = gt_ref[0:1, :] - gt_ref[1:2, :]
    rows = lax.broadcasted_iota(jnp.int32, (2 * DY, 1), 0)
    ab = ab_ref[...]
    comb = comb_ref[...]
    for i in range(G):
        v = g * G + i
        wsel = jnp.where(rows == v, wy,
                         jnp.where(rows == v + DY, delta, 0.0))
        prod = jnp.dot(ab, wsel, preferred_element_type=jnp.float32)
        val_ref[0, i * L:(i + 1) * L, :] = comb + prod


def _sc_body(st_ref, sp_ref, vi_ref, mk_ref, buf, vi_buf, mk_buf, sem):
    c = lax.axis_index("c")
    s = lax.axis_index("s")
    w = s * _NUM_SC_CORES + c
    B = sp_ref.shape[0]
    L, D = buf.shape

    nv = 16
    pltpu.sync_copy(st_ref.at[pl.ds(w, 1)], buf.at[pl.ds(0, 1)])
    row = [buf[0, pl.ds(j * nv, nv)] for j in range(D // nv)]
    wv = jnp.broadcast_to(w, (nv,))
    ones = jnp.ones((nv,), jnp.int32)
    chunk = 16

    def _fill(i, carry):
        r0 = i * chunk
        for r in range(chunk):
            for j in range(D // nv):
                buf[r0 + r, pl.ds(j * nv, nv)] = row[j]
        vi_buf[pl.ds(i * nv, nv)] = wv
        mk_buf[pl.ds(i * nv, nv)] = ones
        return carry

    lax.fori_loop(0, L // chunk, _fill, 0)

    copies = []
    for b in range(B):
        base = w * L
        copies.append(pltpu.async_copy(buf, sp_ref.at[b, pl.ds(base, L)], sem))
        copies.append(
            pltpu.async_copy(vi_buf, vi_ref.at[b, pl.ds(base, L)], sem))
        copies.append(
            pltpu.async_copy(mk_buf, mk_ref.at[b, pl.ds(base, L)], sem))
    for cp in copies:
        cp.wait()


def kernel(y, x, local_table, given_table, space_table, t2v_w, t2v_b,
           vt_w, vt_b):
    B, L, DY = y.shape
    DX = x.shape[-1]
    D = local_table.shape[-1]
    K = t2v_w.shape[-1]
    F = DX * K
    T = DY * L

    G = 8
    val = pl.pallas_call(
        _tc_body,
        grid=(B, DY // G),
        in_specs=[
            pl.BlockSpec(memory_space=pl.ANY),
            pl.BlockSpec(memory_space=pl.ANY),
            pl.BlockSpec((L, D), lambda b, g: (0, 0)),
            pl.BlockSpec((2, D), lambda b, g: (0, 0)),
            pl.BlockSpec((DX, K), lambda b, g: (0, 0)),
            pl.BlockSpec((DX, K), lambda b, g: (0, 0)),
            pl.BlockSpec(memory_space=pl.ANY),
            pl.BlockSpec((1, D), lambda b, g: (0, 0)),
        ],
        out_specs=pl.BlockSpec((1, G * L, D), lambda b, g: (b, g, 0)),
        out_shape=jax.ShapeDtypeStruct((B, T, D), jnp.float32),
        scratch_shapes=[pltpu.VMEM((L, D), jnp.float32),
                        pltpu.VMEM((L, 2 * DY), jnp.float32),
                        pltpu.VMEM((DX, F), jnp.float32),
                        pltpu.VMEM((1, F), jnp.float32),
                        pltpu.VMEM((F + 1, D), jnp.float32),
                        pltpu.VMEM((L, DY), jnp.float32),
                        pltpu.VMEM((L, DX), jnp.float32),
                        pltpu.VMEM((D, F + 1), jnp.float32),
                        pltpu.SemaphoreType.DMA],
        compiler_params=pltpu.CompilerParams(
            dimension_semantics=("arbitrary", "arbitrary")),
    )(y, x, local_table, given_table, t2v_w, t2v_b, vt_w,
      vt_b.reshape(1, D))

    sp, var_idx, mask = pl.kernel(
        _sc_body,
        out_type=[
            jax.ShapeDtypeStruct((B, T, D), jnp.float32),
            jax.ShapeDtypeStruct((B, T), jnp.int32),
            jax.ShapeDtypeStruct((B, T), jnp.int32),
        ],
        mesh=plsc.VectorSubcoreMesh(core_axis_name="c", subcore_axis_name="s"),
        scratch_types=[
            pltpu.VMEM((L, D), jnp.float32),
            pltpu.VMEM((L,), jnp.int32),
            pltpu.VMEM((L,), jnp.int32),
            pltpu.SemaphoreType.DMA,
        ],
    )(space_table)

    return (val, sp, var_idx, mask)


# speedup vs baseline: 1.2157x; 1.2157x over previous
import jax
import jax.numpy as jnp
from jax import lax
from jax.experimental import pallas as pl
from jax.experimental.pallas import tpu as pltpu
from jax.experimental.pallas import tpu_sc as plsc

_NUM_SC_CORES = 2
_NUM_SC_SUBCORES = 16


def _tc_body(y_ref, x_ref, lt_ref, gt_ref, w_ref, b_ref, vtw_ref, vtb_ref,
             val_ref, comb_ref, ab_ref, ew_ref, bf_ref, vtwt_ref):
    b = pl.program_id(0)
    g = pl.program_id(1)
    L, D = comb_ref.shape
    DY = y_ref.shape[2]
    DX, K = w_ref.shape
    F = DX * K
    G = val_ref.shape[1] // L

    @pl.when((b == 0) & (g == 0))
    def _prep_weights():
        fidx = lax.broadcasted_iota(jnp.int32, (DX, F), 1)
        didx = lax.broadcasted_iota(jnp.int32, (DX, F), 0)
        e = (fidx // K == didx).astype(jnp.float32)
        wt = jnp.concatenate([w_ref[...]] * DX, axis=1)
        bt = jnp.concatenate([b_ref[...]] * DX, axis=1)
        ew_ref[...] = e * wt
        bf_ref[...] = jnp.sum(e * bt, axis=0, keepdims=True)
        eye = (lax.broadcasted_iota(jnp.int32, (D, D), 0)
               == lax.broadcasted_iota(jnp.int32, (D, D), 1)
               ).astype(jnp.float32)
        vtwt_ref[0:F + 1, :] = lax.dot_general(
            vtw_ref[...], eye, (((0,), (0,)), ((), ())),
            preferred_element_type=jnp.float32)

    @pl.when(g == 0)
    def _compute_combined():
        xb = x_ref[0]
        xb = jnp.where(jnp.isnan(xb), 0.0, xb)
        xa = jnp.dot(xb, ew_ref[...],
                     preferred_element_type=jnp.float32) + bf_ref[...]
        ksel = (lax.broadcasted_iota(jnp.int32, (1, F), 1) % K) > 0
        feats = jnp.where(ksel, jnp.sin(xa), xa)
        tp = jnp.dot(feats, vtwt_ref[0:F, :],
                     preferred_element_type=jnp.float32)
        comb_ref[...] = (tp + lt_ref[...] + vtb_ref[...] + gt_ref[1:2, :])
        y2d = y_ref[0]
        nan2d = jnp.isnan(y2d)
        ab_ref[:, :DY] = jnp.where(nan2d, 0.0, y2d)
        ab_ref[:, DY:] = nan2d.astype(jnp.float32)

    wy = vtwt_ref[F:F + 1, :]
    delta = gt_ref[0:1, :] - gt_ref[1:2, :]
    rows = lax.broadcasted_iota(jnp.int32, (2 * DY, 1), 0)
    ab = ab_ref[...]
    comb = comb_ref[...]
    for i in range(G):
        v = g * G + i
        wsel = jnp.where(rows == v, wy,
                         jnp.where(rows == v + DY, delta, 0.0))
        prod = jnp.dot(ab, wsel, preferred_element_type=jnp.float32)
        val_ref[0, i * L:(i + 1) * L, :] = comb + prod


def _sc_body(st_ref, sp_ref, vi_ref, mk_ref, buf, vi_buf, mk_buf, sem):
    c = lax.axis_index("c")
    s = lax.axis_index("s")
    w = s * _NUM_SC_CORES + c
    B = sp_ref.shape[0]
    L, D = buf.shape

    nv = 16
    pltpu.sync_copy(st_ref.at[pl.ds(w, 1)], buf.at[pl.ds(0, 1)])
    row = [buf[0, pl.ds(j * nv, nv)] for j in range(D // nv)]
    wv = jnp.broadcast_to(w, (nv,))
    ones = jnp.ones((nv,), jnp.int32)
    chunk = 16

    def _fill(i, carry):
        r0 = i * chunk
        for r in range(chunk):
            for j in range(D // nv):
                buf[r0 + r, pl.ds(j * nv, nv)] = row[j]
        vi_buf[pl.ds(i * nv, nv)] = wv
        mk_buf[pl.ds(i * nv, nv)] = ones
        return carry

    lax.fori_loop(0, L // chunk, _fill, 0)

    copies = []
    for b in range(B):
        base = w * L
        copies.append(pltpu.async_copy(buf, sp_ref.at[b, pl.ds(base, L)], sem))
        copies.append(
            pltpu.async_copy(vi_buf, vi_ref.at[b, pl.ds(base, L)], sem))
        copies.append(
            pltpu.async_copy(mk_buf, mk_ref.at[b, pl.ds(base, L)], sem))
    for cp in copies:
        cp.wait()


def kernel(y, x, local_table, given_table, space_table, t2v_w, t2v_b,
           vt_w, vt_b):
    B, L, DY = y.shape
    DX = x.shape[-1]
    D = local_table.shape[-1]
    K = t2v_w.shape[-1]
    F = DX * K
    T = DY * L

    G = 16
    val = pl.pallas_call(
        _tc_body,
        grid=(B, DY // G),
        in_specs=[
            pl.BlockSpec((1, L, DY), lambda b, g: (b, 0, 0)),
            pl.BlockSpec((1, L, DX), lambda b, g: (b, 0, 0)),
            pl.BlockSpec((L, D), lambda b, g: (0, 0)),
            pl.BlockSpec((2, D), lambda b, g: (0, 0)),
            pl.BlockSpec((DX, K), lambda b, g: (0, 0)),
            pl.BlockSpec((DX, K), lambda b, g: (0, 0)),
            pl.BlockSpec((D, F + 1), lambda b, g: (0, 0)),
            pl.BlockSpec((1, D), lambda b, g: (0, 0)),
        ],
        out_specs=pl.BlockSpec((1, G * L, D), lambda b, g: (b, g, 0)),
        out_shape=jax.ShapeDtypeStruct((B, T, D), jnp.float32),
        scratch_shapes=[pltpu.VMEM((L, D), jnp.float32),
                        pltpu.VMEM((L, 2 * DY), jnp.float32),
                        pltpu.VMEM((DX, F), jnp.float32),
                        pltpu.VMEM((1, F), jnp.float32),
                        pltpu.VMEM((F + 1, D), jnp.float32)],
        compiler_params=pltpu.CompilerParams(
            dimension_semantics=("arbitrary", "arbitrary")),
    )(y, x, local_table, given_table, t2v_w, t2v_b, vt_w,
      vt_b.reshape(1, D))

    sp, var_idx, mask = pl.kernel(
        _sc_body,
        out_type=[
            jax.ShapeDtypeStruct((B, T, D), jnp.float32),
            jax.ShapeDtypeStruct((B, T), jnp.int32),
            jax.ShapeDtypeStruct((B, T), jnp.int32),
        ],
        mesh=plsc.VectorSubcoreMesh(core_axis_name="c", subcore_axis_name="s"),
        scratch_types=[
            pltpu.VMEM((L, D), jnp.float32),
            pltpu.VMEM((L,), jnp.int32),
            pltpu.VMEM((L,), jnp.int32),
            pltpu.SemaphoreType.DMA,
        ],
    )(space_table)

    return (val, sp, var_idx, mask)


# speedup vs baseline: 1.2829x; 1.0552x over previous
import jax
import jax.numpy as jnp
from jax import lax
from jax.experimental import pallas as pl
from jax.experimental.pallas import tpu as pltpu
from jax.experimental.pallas import tpu_sc as plsc

_NUM_SC_CORES = 2
_NUM_SC_SUBCORES = 16


def _tc_body(y_ref, x_ref, lt_ref, gt_ref, w_ref, b_ref, vtw_ref, vtb_ref,
             val_ref, comb_ref, ab_ref, ew_ref, bf_ref, vtwt_ref):
    b = pl.program_id(0)
    g = pl.program_id(1)
    L, D = comb_ref.shape
    DY = y_ref.shape[2]
    DX, K = w_ref.shape
    F = DX * K
    G = val_ref.shape[1] // L

    @pl.when((b == 0) & (g == 0))
    def _prep_weights():
        fidx = lax.broadcasted_iota(jnp.int32, (DX, F), 1)
        didx = lax.broadcasted_iota(jnp.int32, (DX, F), 0)
        e = (fidx // K == didx).astype(jnp.float32)
        wt = jnp.concatenate([w_ref[...]] * DX, axis=1)
        bt = jnp.concatenate([b_ref[...]] * DX, axis=1)
        ew_ref[...] = e * wt
        bf_ref[...] = jnp.sum(e * bt, axis=0, keepdims=True)
        eye = (lax.broadcasted_iota(jnp.int32, (D, D), 0)
               == lax.broadcasted_iota(jnp.int32, (D, D), 1)
               ).astype(jnp.float32)
        vtwt_ref[0:F + 1, :] = lax.dot_general(
            vtw_ref[...], eye, (((0,), (0,)), ((), ())),
            preferred_element_type=jnp.float32)

    @pl.when(g == 0)
    def _compute_combined():
        xb = x_ref[0]
        xb = jnp.where(jnp.isnan(xb), 0.0, xb)
        xa = jnp.dot(xb, ew_ref[...],
                     preferred_element_type=jnp.float32) + bf_ref[...]
        ksel = (lax.broadcasted_iota(jnp.int32, (1, F), 1) % K) > 0
        feats = jnp.where(ksel, jnp.sin(xa), xa)
        tp = jnp.dot(feats, vtwt_ref[0:F, :],
                     preferred_element_type=jnp.float32)
        comb_ref[...] = (tp + lt_ref[...] + vtb_ref[...] + gt_ref[1:2, :])
        y2d = y_ref[0]
        nan2d = jnp.isnan(y2d)
        ab_ref[:, :DY] = jnp.where(nan2d, 0.0, y2d)
        ab_ref[:, DY:] = nan2d.astype(jnp.float32)

    wy = vtwt_ref[F:F + 1, :]
    delta = gt_ref[0:1, :] - gt_ref[1:2, :]
    rows = lax.broadcasted_iota(jnp.int32, (2 * DY, 1), 0)
    ab = ab_ref[...]
    comb = comb_ref[...]
    for i in range(G):
        v = g * G + i
        wsel = jnp.where(rows == v, wy,
                         jnp.where(rows == v + DY, delta, 0.0))
        prod = jnp.dot(ab, wsel, preferred_element_type=jnp.float32)
        val_ref[0, i * L:(i + 1) * L, :] = comb + prod


def _sc_body(st_ref, sp_ref, vi_ref, mk_ref, buf, vi_buf, mk_buf, sem):
    c = lax.axis_index("c")
    s = lax.axis_index("s")
    w = s * _NUM_SC_CORES + c
    B = sp_ref.shape[0]
    L, D = buf.shape

    nv = 16
    pltpu.sync_copy(st_ref.at[pl.ds(w, 1)], buf.at[pl.ds(0, 1)])
    row = [buf[0, pl.ds(j * nv, nv)] for j in range(D // nv)]
    wv = jnp.broadcast_to(w, (nv,))
    ones = jnp.ones((nv,), jnp.int32)
    chunk = 16

    def _fill(i, carry):
        r0 = i * chunk
        for r in range(chunk):
            for j in range(D // nv):
                buf[r0 + r, pl.ds(j * nv, nv)] = row[j]
        vi_buf[pl.ds(i * nv, nv)] = wv
        mk_buf[pl.ds(i * nv, nv)] = ones
        return carry

    lax.fori_loop(0, L // chunk, _fill, 0)

    copies = []
    for b in range(B):
        base = w * L
        copies.append(pltpu.async_copy(buf, sp_ref.at[b, pl.ds(base, L)], sem))
        copies.append(
            pltpu.async_copy(vi_buf, vi_ref.at[b, pl.ds(base, L)], sem))
        copies.append(
            pltpu.async_copy(mk_buf, mk_ref.at[b, pl.ds(base, L)], sem))
    for cp in copies:
        cp.wait()


def kernel(y, x, local_table, given_table, space_table, t2v_w, t2v_b,
           vt_w, vt_b):
    B, L, DY = y.shape
    DX = x.shape[-1]
    D = local_table.shape[-1]
    K = t2v_w.shape[-1]
    F = DX * K
    T = DY * L

    G = 32
    val = pl.pallas_call(
        _tc_body,
        grid=(B, DY // G),
        in_specs=[
            pl.BlockSpec((1, L, DY), lambda b, g: (b, 0, 0)),
            pl.BlockSpec((1, L, DX), lambda b, g: (b, 0, 0)),
            pl.BlockSpec((L, D), lambda b, g: (0, 0)),
            pl.BlockSpec((2, D), lambda b, g: (0, 0)),
            pl.BlockSpec((DX, K), lambda b, g: (0, 0)),
            pl.BlockSpec((DX, K), lambda b, g: (0, 0)),
            pl.BlockSpec((D, F + 1), lambda b, g: (0, 0)),
            pl.BlockSpec((1, D), lambda b, g: (0, 0)),
        ],
        out_specs=pl.BlockSpec((1, G * L, D), lambda b, g: (b, g, 0)),
        out_shape=jax.ShapeDtypeStruct((B, T, D), jnp.float32),
        scratch_shapes=[pltpu.VMEM((L, D), jnp.float32),
                        pltpu.VMEM((L, 2 * DY), jnp.float32),
                        pltpu.VMEM((DX, F), jnp.float32),
                        pltpu.VMEM((1, F), jnp.float32),
                        pltpu.VMEM((F + 1, D), jnp.float32)],
        compiler_params=pltpu.CompilerParams(
            dimension_semantics=("arbitrary", "arbitrary")),
    )(y, x, local_table, given_table, t2v_w, t2v_b, vt_w,
      vt_b.reshape(1, D))

    sp, var_idx, mask = pl.kernel(
        _sc_body,
        out_type=[
            jax.ShapeDtypeStruct((B, T, D), jnp.float32),
            jax.ShapeDtypeStruct((B, T), jnp.int32),
            jax.ShapeDtypeStruct((B, T), jnp.int32),
        ],
        mesh=plsc.VectorSubcoreMesh(core_axis_name="c", subcore_axis_name="s"),
        scratch_types=[
            pltpu.VMEM((L, D), jnp.float32),
            pltpu.VMEM((L,), jnp.int32),
            pltpu.VMEM((L,), jnp.int32),
            pltpu.SemaphoreType.DMA,
        ],
    )(space_table)

    return (val, sp, var_idx, mask)
